# baseline (speedup 1.0000x reference)
import functools
import math

import jax
import jax.numpy as jnp
from jax import lax
from jax.experimental import pallas as pl
from jax.experimental.pallas import tpu as pltpu
from jax.experimental.pallas import tpu_sc as plsc

_NSPEC = 10
_RB = 8
_CUT = 5.0
_P = 2.0
_AVG = 32.0
_INV_SQRT_AVG = 1.0 / math.sqrt(_AVG)

_NC = 2
_NS = 16
_K = 128


def _edge_body(dispT_ref, r0c, r1b, r2b, h_ref):
    dT = dispT_ref[...]
    sq = (dT[0:1, :] * dT[0:1, :] + dT[1:2, :] * dT[1:2, :]
          + dT[2:3, :] * dT[2:3, :])
    r = jnp.where(sq == 0.0, 0.0, jnp.sqrt(sq))
    B = r.shape[1]
    r8 = jnp.broadcast_to(r, (_RB, B))
    w = (lax.broadcasted_iota(jnp.int32, (_RB, 1), 0).astype(jnp.float32)
         + 1.0) * jnp.pi
    rbT = (2.0 / _CUT) * jnp.where(r8 == 0.0, w / _CUT,
                                   jnp.sin(w * r8 / _CUT) / r8)
    x = r / _CUT
    x2 = x * x
    cut = (1.0 - 6.0 * x2 + 8.0 * x2 * x - 3.0 * x2 * x2)
    cut = cut * jnp.where(x < 1.0, 1.0, 0.0)
    rbT = (rbT * cut).astype(jnp.bfloat16)

    t = lax.dot_general(rbT, r0c[...], (((0,), (0,)), ((), ())),
                        preferred_element_type=jnp.float32)
    t = jax.nn.silu(t).astype(jnp.bfloat16)
    t = jax.nn.silu(jnp.dot(t, r1b[...], preferred_element_type=jnp.float32))
    h_ref[...] = jnp.dot(t.astype(jnp.bfloat16), r2b[...],
                         preferred_element_type=jnp.float32)


def _edge_mlps(disp, r00, r01, r02p, r10, r11, r12):
    E = disp.shape[0]
    B = next(b for b in (5120, 2560, 1280, 640, 128) if E % b == 0)
    r0c = jnp.concatenate([r00, r10], axis=1).astype(jnp.bfloat16)
    z64 = jnp.zeros((64, 64), jnp.float32)
    r1b = jnp.block([[r01, z64], [z64, r11]]).astype(jnp.bfloat16)
    r2b = jnp.block(
        [[r02p, jnp.zeros((64, 64), jnp.float32),
          jnp.zeros((64, 48), jnp.float32)],
         [jnp.zeros((64, 16), jnp.float32), r12,
          jnp.zeros((64, 48), jnp.float32)]]).astype(jnp.bfloat16)
    dispT = disp.T

    def full(shape):
        return pl.BlockSpec(shape, lambda i: (0,) * len(shape))

    return pl.pallas_call(
        _edge_body,
        grid=(E // B,),
        in_specs=[
            pl.BlockSpec((3, B), lambda i: (0, i)),
            full((_RB, 128)), full((128, 128)), full((128, 128)),
        ],
        out_specs=pl.BlockSpec((B, 128), lambda i: (i, 0)),
        out_shape=jax.ShapeDtypeStruct((E, 128), jnp.float32),
        compiler_params=pltpu.CompilerParams(
            dimension_semantics=("parallel",)),
    )(dispT, r0c, r1b, r2b)


_R = 4


def _sc_gather_scatter(table, h, col_off, D, Np, snd3, rcv3):
    E = h.shape[0]
    NW = _NC * _NS
    epw = E // NW
    nchunks = epw // _K
    ngroups = nchunks // _R
    rpt = Np // _NS
    mesh = plsc.VectorSubcoreMesh(core_axis_name="c", subcore_axis_name="s")

    @functools.partial(
        pl.kernel,
        out_type=jax.ShapeDtypeStruct((_NC * Np, D), jnp.float32),
        mesh=mesh,
        scratch_types=[
            pltpu.VMEM((nchunks, _K), jnp.int32),
            pltpu.VMEM((nchunks, _K), jnp.int32),
            pltpu.VMEM((_R, _K, D), jnp.float32),
            pltpu.VMEM((_R * _K, D), jnp.float32),
            pltpu.VMEM_SHARED((Np, D), jnp.float32),
            pltpu.SemaphoreType.DMA((_R,)),
            pltpu.SemaphoreType.DMA,
            pltpu.SemaphoreType.DMA((_R,)),
        ],
        compiler_params=pltpu.CompilerParams(use_tc_tiling_on_sc=False),
    )
    def k(table_hbm, h_hbm, snd_hbm, rcv_hbm, out_hbm,
          snd_v, rcv_v, g_v, h_v, acc, sem_g, sem_h, sem_s):
        cid = lax.axis_index("c")
        sid = lax.axis_index("s")
        wid = cid * _NS + sid
        pltpu.sync_copy(snd_hbm.at[wid], snd_v)
        pltpu.sync_copy(rcv_hbm.at[wid], rcv_v)
        zrows = _R * _K

        @plsc.parallel_loop(0, zrows, 1, unroll=8)
        def zrow(i):
            for j in range(D // 16):
                h_v[i, pl.ds(j * 16, 16)] = jnp.zeros((16,), jnp.float32)

        off = 0
        while off < rpt:
            n = min(zrows, rpt - off)
            pltpu.sync_copy(h_v.at[pl.ds(0, n)],
                            acc.at[pl.ds(sid * rpt + off, n)])
            off += n
        plsc.subcore_barrier()

        def group(gi, carry):
            cb = gi * _R
            hbase = pl.multiple_of(wid * epw + cb * _K, 8)
            dh = pltpu.async_copy(
                h_hbm.at[pl.ds(hbase, _R * _K), pl.ds(col_off, D)], h_v, sem_h)
            gds = [
                pltpu.async_copy(table_hbm.at[snd_v.at[cb + r]], g_v.at[r],
                                 sem_g.at[r])
                for r in range(_R)
            ]
            dh.wait()
            sds = []
            for r in range(_R):
                gds[r].wait()

                @plsc.parallel_loop(0, _K, 1, unroll=4)
                def mul_row(i):
                    for j in range(D // 16):
                        sl = pl.ds(j * 16, 16)
                        g_v[r, i, sl] = g_v[r, i, sl] * h_v[r * _K + i, sl]

                sds.append(pltpu.async_copy(g_v.at[r], acc.at[rcv_v.at[cb + r]],
                                            sem_s.at[r], add=True))
            for d in sds:
                d.wait()
            return carry

        lax.fori_loop(0, ngroups, group, 0)
        plsc.subcore_barrier()
        pltpu.sync_copy(acc.at[pl.ds(sid * rpt, rpt)],
                        out_hbm.at[pl.ds(cid * Np + sid * rpt, rpt)])

    return k(table, h, snd3, rcv3)


def _node0_body(spec_ref, w1p, d0p, oh_ref, t0_ref, sk_ref):
    s = spec_ref[...]
    B = s.shape[0]
    oh = jnp.where(lax.broadcasted_iota(jnp.int32, (B, 16), 1) == s,
                   1.0, 0.0)
    oh_ref[...] = oh
    t0_ref[...] = jnp.dot(oh, w1p[...], preferred_element_type=jnp.float32)
    sk_ref[...] = jnp.dot(oh, d0p[...], preferred_element_type=jnp.float32)


def _node0(spec2d, w1p, d0p):
    N = spec2d.shape[0]
    B = 2000

    def full(shape):
        return pl.BlockSpec(shape, lambda i: (0,) * len(shape))

    return pl.pallas_call(
        _node0_body,
        grid=(N // B,),
        in_specs=[
            pl.BlockSpec((B, 1), lambda i: (i, 0)),
            full((16, 16)), full((16, 64)),
        ],
        out_specs=[pl.BlockSpec((B, 16), lambda i: (i, 0)),
                   pl.BlockSpec((B, 16), lambda i: (i, 0)),
                   pl.BlockSpec((B, 64), lambda i: (i, 0))],
        out_shape=[jax.ShapeDtypeStruct((N, 16), jnp.float32),
                   jax.ShapeDtypeStruct((N, 16), jnp.float32),
                   jax.ShapeDtypeStruct((N, 64), jnp.float32)],
        compiler_params=pltpu.CompilerParams(
            dimension_semantics=("parallel",)),
    )(spec2d, w1p, d0p)


def _node1_body(p0, p1, skip0, oh, w2, w1n, wsk, t1_ref, sk_ref):
    agg = (p0[...] + p1[...]) * _INV_SQRT_AVG
    f1 = jax.nn.silu(
        jnp.dot(agg, w2[...], preferred_element_type=jnp.float32) + skip0[...])
    t1_ref[...] = jnp.dot(f1, w1n[...], preferred_element_type=jnp.float32)
    acc = jnp.zeros_like(sk_ref)
    for s in range(_NSPEC):
        acc = acc + jnp.dot(f1 * oh[:, s:s + 1], wsk[s],
                            preferred_element_type=jnp.float32)
    sk_ref[...] = acc


def _node1(p0, p1, skip0, oh, w2, w1n, wsk):
    N = p0.shape[0]
    B = 2000

    def full(shape):
        return pl.BlockSpec(shape, lambda i: (0,) * len(shape))

    return pl.pallas_call(
        _node1_body,
        grid=(N // B,),
        in_specs=[
            pl.BlockSpec((B, 16), lambda i: (i, 0)),
            pl.BlockSpec((B, 16), lambda i: (i, 0)),
            pl.BlockSpec((B, 64), lambda i: (i, 0)),
            pl.BlockSpec((B, 16), lambda i: (i, 0)),
            full((16, 64)), full((64, 64)), full((_NSPEC, 64, 64)),
        ],
        out_specs=[pl.BlockSpec((B, 64), lambda i: (i, 0)),
                   pl.BlockSpec((B, 64), lambda i: (i, 0))],
        out_shape=[jax.ShapeDtypeStruct((N, 64), jnp.float32),
                   jax.ShapeDtypeStruct((N, 64), jnp.float32)],
        compiler_params=pltpu.CompilerParams(
            dimension_semantics=("parallel",)),
    )(p0, p1, skip0, oh, w2, w1n, wsk)


def _node2_body(p0, p1, skip1, oh, w2, wr, aep, out_ref):
    agg = (p0[...] + p1[...]) * _INV_SQRT_AVG
    f2 = jax.nn.silu(
        jnp.dot(agg, w2[...], preferred_element_type=jnp.float32) + skip1[...])
    out_ref[...] = (jnp.dot(f2, wr[...], preferred_element_type=jnp.float32)
                    + jnp.dot(oh[...], aep[...],
                              preferred_element_type=jnp.float32))


def _node2(p0, p1, skip1, oh, w2, wr, aep):
    N = p0.shape[0]
    B = 2000

    def full(shape):
        return pl.BlockSpec(shape, lambda i: (0,) * len(shape))

    return pl.pallas_call(
        _node2_body,
        grid=(N // B,),
        in_specs=[
            pl.BlockSpec((B, 64), lambda i: (i, 0)),
            pl.BlockSpec((B, 64), lambda i: (i, 0)),
            pl.BlockSpec((B, 64), lambda i: (i, 0)),
            pl.BlockSpec((B, 16), lambda i: (i, 0)),
            full((64, 64)), full((64, 1)), full((16, 1)),
        ],
        out_specs=pl.BlockSpec((B, 1), lambda i: (i, 0)),
        out_shape=jax.ShapeDtypeStruct((N, 1), jnp.float32),
        compiler_params=pltpu.CompilerParams(
            dimension_semantics=("parallel",)),
    )(p0, p1, skip1, oh, w2, wr, aep)


def kernel(displacements, species, senders, receivers, params):
    L0, L1 = params["layers"]
    N = species.shape[0]
    E = senders.shape[0]
    spec = species.astype(jnp.int32)
    NW = _NC * _NS
    Np = -(-N // (8 * _NS)) * (8 * _NS)
    cpw = -(-E // (NW * _K))
    cpw = -(-cpw // _R) * _R
    EP = NW * _K * cpw
    snd = jnp.pad(senders.astype(jnp.int32), (0, EP - E)
                  ).reshape(NW, cpw, _K)
    rcv = jnp.pad(receivers.astype(jnp.int32), (0, EP - E),
                  constant_values=Np - 1).reshape(NW, cpw, _K)
    dpad = jnp.pad(displacements, ((0, EP - E), (0, 0)))

    r02p = jnp.pad(L0["R"][2], ((0, 0), (0, 16 - _NSPEC)))
    h_all = _edge_mlps(dpad, L0["R"][0], L0["R"][1], r02p,
                       L1["R"][0], L1["R"][1], L1["R"][2])

    pad6 = (0, 16 - _NSPEC)
    w1p = jnp.pad(L0["W1"], (pad6, pad6))
    diag0 = L0["Wskip"][jnp.arange(_NSPEC), jnp.arange(_NSPEC)]
    d0p = jnp.pad(diag0, (pad6, (0, 0)))
    oh, table0, skip0 = _node0(spec.reshape(N, 1), w1p, d0p)
    parts0 = _sc_gather_scatter(table0, h_all, 0, 16, Np, snd, rcv)
    w2_0p = jnp.pad(L0["W2"], (pad6, (0, 0)))

    table1, skip1 = _node1(parts0[:N], parts0[Np:Np + N], skip0, oh,
                           w2_0p, L1["W1"], L1["Wskip"])
    parts1 = _sc_gather_scatter(table1, h_all, 16, 64, Np, snd, rcv)
    aep = jnp.pad(params["atom_energies"], pad6).reshape(16, 1)
    node_e = _node2(parts1[:N], parts1[Np:Np + N], skip1, oh, L1["W2"],
                    params["Wr"], aep)[:, 0]
    return node_e


# speedup vs baseline: 1.5063x; 1.5063x over previous
import functools
import math

import jax
import jax.numpy as jnp
from jax import lax
from jax.experimental import pallas as pl
from jax.experimental.pallas import tpu as pltpu
from jax.experimental.pallas import tpu_sc as plsc

_NSPEC = 10
_RB = 8
_CUT = 5.0
_P = 2.0
_AVG = 32.0
_INV_SQRT_AVG = 1.0 / math.sqrt(_AVG)

_NC = 2
_NS = 16
_K = 80


def _edge_body(dispT_ref, r0c, r1b, r2b, h_ref):
    dT = dispT_ref[...]
    sq = (dT[0:1, :] * dT[0:1, :] + dT[1:2, :] * dT[1:2, :]
          + dT[2:3, :] * dT[2:3, :])
    r = jnp.where(sq == 0.0, 0.0, jnp.sqrt(sq))
    B = r.shape[1]
    r8 = jnp.broadcast_to(r, (_RB, B))
    w = (lax.broadcasted_iota(jnp.int32, (_RB, 1), 0).astype(jnp.float32)
         + 1.0) * jnp.pi
    rbT = (2.0 / _CUT) * jnp.where(r8 == 0.0, w / _CUT,
                                   jnp.sin(w * r8 / _CUT) / r8)
    x = r / _CUT
    x2 = x * x
    cut = (1.0 - 6.0 * x2 + 8.0 * x2 * x - 3.0 * x2 * x2)
    cut = cut * jnp.where(x < 1.0, 1.0, 0.0)
    rbT = (rbT * cut).astype(jnp.bfloat16)

    t = lax.dot_general(rbT, r0c[...], (((0,), (0,)), ((), ())),
                        preferred_element_type=jnp.float32)
    t = jax.nn.silu(t).astype(jnp.bfloat16)
    t = jax.nn.silu(jnp.dot(t, r1b[...], preferred_element_type=jnp.float32))
    h_ref[...] = jnp.dot(t.astype(jnp.bfloat16), r2b[...],
                         preferred_element_type=jnp.float32)


def _edge_mlps(disp, r00, r01, r02p, r10, r11, r12):
    E = disp.shape[0]
    B = next(b for b in (6400, 5120, 2560, 1280, 640, 128) if E % b == 0)
    r0c = jnp.concatenate([r00, r10], axis=1).astype(jnp.bfloat16)
    z64 = jnp.zeros((64, 64), jnp.float32)
    r1b = jnp.block([[r01, z64], [z64, r11]]).astype(jnp.bfloat16)
    r2b = jnp.block(
        [[r02p, jnp.zeros((64, 64), jnp.float32),
          jnp.zeros((64, 48), jnp.float32)],
         [jnp.zeros((64, 16), jnp.float32), r12,
          jnp.zeros((64, 48), jnp.float32)]]).astype(jnp.bfloat16)
    dispT = disp.T

    def full(shape):
        return pl.BlockSpec(shape, lambda i: (0,) * len(shape))

    return pl.pallas_call(
        _edge_body,
        grid=(E // B,),
        in_specs=[
            pl.BlockSpec((3, B), lambda i: (0, i)),
            full((_RB, 128)), full((128, 128)), full((128, 128)),
        ],
        out_specs=pl.BlockSpec((B, 128), lambda i: (i, 0)),
        out_shape=jax.ShapeDtypeStruct((E, 128), jnp.float32),
        compiler_params=pltpu.CompilerParams(
            dimension_semantics=("parallel",)),
    )(dispT, r0c, r1b, r2b)


_R = 5


def _sc_gather_scatter(table, h, col_off, D, Np, snd3, rcv3):
    E = h.shape[0]
    NW = _NC * _NS
    epw = E // NW
    nchunks = epw // _K
    ngroups = nchunks // _R
    rpt = Np // _NS
    mesh = plsc.VectorSubcoreMesh(core_axis_name="c", subcore_axis_name="s")

    @functools.partial(
        pl.kernel,
        out_type=jax.ShapeDtypeStruct((_NC * Np, D), jnp.float32),
        mesh=mesh,
        scratch_types=[
            pltpu.VMEM((nchunks, _K), jnp.int32),
            pltpu.VMEM((nchunks, _K), jnp.int32),
            pltpu.VMEM((_R, _K, D), jnp.float32),
            pltpu.VMEM((_R * _K, D), jnp.float32),
            pltpu.VMEM_SHARED((Np, D), jnp.float32),
            pltpu.SemaphoreType.DMA((_R,)),
            pltpu.SemaphoreType.DMA,
            pltpu.SemaphoreType.DMA((_R,)),
        ],
        compiler_params=pltpu.CompilerParams(use_tc_tiling_on_sc=False),
    )
    def k(table_hbm, h_hbm, snd_hbm, rcv_hbm, out_hbm,
          snd_v, rcv_v, g_v, h_v, acc, sem_g, sem_h, sem_s):
        cid = lax.axis_index("c")
        sid = lax.axis_index("s")
        wid = cid * _NS + sid
        pltpu.sync_copy(snd_hbm.at[wid], snd_v)
        pltpu.sync_copy(rcv_hbm.at[wid], rcv_v)
        zrows = _R * _K

        @plsc.parallel_loop(0, zrows, 1, unroll=8)
        def zrow(i):
            for j in range(D // 16):
                h_v[i, pl.ds(j * 16, 16)] = jnp.zeros((16,), jnp.float32)

        off = 0
        while off < rpt:
            n = min(zrows, rpt - off)
            pltpu.sync_copy(h_v.at[pl.ds(0, n)],
                            acc.at[pl.ds(sid * rpt + off, n)])
            off += n
        plsc.subcore_barrier()

        def group(gi, carry):
            cb = gi * _R
            hbase = pl.multiple_of(wid * epw + cb * _K, 8)
            dh = pltpu.async_copy(
                h_hbm.at[pl.ds(hbase, _R * _K), pl.ds(col_off, D)], h_v, sem_h)
            gds = [
                pltpu.async_copy(table_hbm.at[snd_v.at[cb + r]], g_v.at[r],
                                 sem_g.at[r])
                for r in range(_R)
            ]
            dh.wait()
            sds = []
            for r in range(_R):
                gds[r].wait()

                @plsc.parallel_loop(0, _K, 1, unroll=4)
                def mul_row(i):
                    for j in range(D // 16):
                        sl = pl.ds(j * 16, 16)
                        g_v[r, i, sl] = g_v[r, i, sl] * h_v[r * _K + i, sl]

                sds.append(pltpu.async_copy(g_v.at[r], acc.at[rcv_v.at[cb + r]],
                                            sem_s.at[r], add=True))
            for d in sds:
                d.wait()
            return carry

        lax.fori_loop(0, ngroups, group, 0)
        plsc.subcore_barrier()
        pltpu.sync_copy(acc.at[pl.ds(sid * rpt, rpt)],
                        out_hbm.at[pl.ds(cid * Np + sid * rpt, rpt)])

    return k(table, h, snd3, rcv3)


def _node0_body(spec_ref, w1p, d0p, oh_ref, t0_ref, sk_ref):
    s = spec_ref[...]
    B = s.shape[0]
    oh = jnp.where(lax.broadcasted_iota(jnp.int32, (B, 16), 1) == s,
                   1.0, 0.0)
    oh_ref[...] = oh
    t0_ref[...] = jnp.dot(oh, w1p[...], preferred_element_type=jnp.float32)
    sk_ref[...] = jnp.dot(oh, d0p[...], preferred_element_type=jnp.float32)


def _node0(spec2d, w1p, d0p):
    N = spec2d.shape[0]
    B = 2000

    def full(shape):
        return pl.BlockSpec(shape, lambda i: (0,) * len(shape))

    return pl.pallas_call(
        _node0_body,
        grid=(N // B,),
        in_specs=[
            pl.BlockSpec((B, 1), lambda i: (i, 0)),
            full((16, 16)), full((16, 64)),
        ],
        out_specs=[pl.BlockSpec((B, 16), lambda i: (i, 0)),
                   pl.BlockSpec((B, 16), lambda i: (i, 0)),
                   pl.BlockSpec((B, 64), lambda i: (i, 0))],
        out_shape=[jax.ShapeDtypeStruct((N, 16), jnp.float32),
                   jax.ShapeDtypeStruct((N, 16), jnp.float32),
                   jax.ShapeDtypeStruct((N, 64), jnp.float32)],
        compiler_params=pltpu.CompilerParams(
            dimension_semantics=("parallel",)),
    )(spec2d, w1p, d0p)


def _node1_body(p0, p1, skip0, oh, w2, w1n, wsk, t1_ref, sk_ref):
    agg = (p0[...] + p1[...]) * _INV_SQRT_AVG
    f1 = jax.nn.silu(
        jnp.dot(agg, w2[...], preferred_element_type=jnp.float32) + skip0[...])
    t1_ref[...] = jnp.dot(f1, w1n[...], preferred_element_type=jnp.float32)
    acc = jnp.zeros_like(sk_ref)
    for s in range(_NSPEC):
        acc = acc + jnp.dot(f1 * oh[:, s:s + 1], wsk[s],
                            preferred_element_type=jnp.float32)
    sk_ref[...] = acc


def _node1(p0, p1, skip0, oh, w2, w1n, wsk):
    N = p0.shape[0]
    B = 2000

    def full(shape):
        return pl.BlockSpec(shape, lambda i: (0,) * len(shape))

    return pl.pallas_call(
        _node1_body,
        grid=(N // B,),
        in_specs=[
            pl.BlockSpec((B, 16), lambda i: (i, 0)),
            pl.BlockSpec((B, 16), lambda i: (i, 0)),
            pl.BlockSpec((B, 64), lambda i: (i, 0)),
            pl.BlockSpec((B, 16), lambda i: (i, 0)),
            full((16, 64)), full((64, 64)), full((_NSPEC, 64, 64)),
        ],
        out_specs=[pl.BlockSpec((B, 64), lambda i: (i, 0)),
                   pl.BlockSpec((B, 64), lambda i: (i, 0))],
        out_shape=[jax.ShapeDtypeStruct((N, 64), jnp.float32),
                   jax.ShapeDtypeStruct((N, 64), jnp.float32)],
        compiler_params=pltpu.CompilerParams(
            dimension_semantics=("parallel",)),
    )(p0, p1, skip0, oh, w2, w1n, wsk)


def _node2_body(p0, p1, skip1, oh, w2, wr, aep, out_ref):
    agg = (p0[...] + p1[...]) * _INV_SQRT_AVG
    f2 = jax.nn.silu(
        jnp.dot(agg, w2[...], preferred_element_type=jnp.float32) + skip1[...])
    out_ref[...] = (jnp.dot(f2, wr[...], preferred_element_type=jnp.float32)
                    + jnp.dot(oh[...], aep[...],
                              preferred_element_type=jnp.float32))


def _node2(p0, p1, skip1, oh, w2, wr, aep):
    N = p0.shape[0]
    B = 2000

    def full(shape):
        return pl.BlockSpec(shape, lambda i: (0,) * len(shape))

    return pl.pallas_call(
        _node2_body,
        grid=(N // B,),
        in_specs=[
            pl.BlockSpec((B, 64), lambda i: (i, 0)),
            pl.BlockSpec((B, 64), lambda i: (i, 0)),
            pl.BlockSpec((B, 64), lambda i: (i, 0)),
            pl.BlockSpec((B, 16), lambda i: (i, 0)),
            full((64, 64)), full((64, 1)), full((16, 1)),
        ],
        out_specs=pl.BlockSpec((B, 1), lambda i: (i, 0)),
        out_shape=jax.ShapeDtypeStruct((N, 1), jnp.float32),
        compiler_params=pltpu.CompilerParams(
            dimension_semantics=("parallel",)),
    )(p0, p1, skip1, oh, w2, wr, aep)


def kernel(displacements, species, senders, receivers, params):
    L0, L1 = params["layers"]
    N = species.shape[0]
    E = senders.shape[0]
    spec = species.astype(jnp.int32)
    NW = _NC * _NS
    Np = -(-N // (8 * _NS)) * (8 * _NS)
    cpw = -(-E // (NW * _K))
    cpw = -(-cpw // _R) * _R
    EP = NW * _K * cpw
    snd = jnp.pad(senders.astype(jnp.int32), (0, EP - E)
                  ).reshape(NW, cpw, _K)
    rcv = jnp.pad(receivers.astype(jnp.int32), (0, EP - E),
                  constant_values=Np - 1).reshape(NW, cpw, _K)
    dpad = jnp.pad(displacements, ((0, EP - E), (0, 0)))

    r02p = jnp.pad(L0["R"][2], ((0, 0), (0, 16 - _NSPEC)))
    h_all = _edge_mlps(dpad, L0["R"][0], L0["R"][1], r02p,
                       L1["R"][0], L1["R"][1], L1["R"][2])

    pad6 = (0, 16 - _NSPEC)
    w1p = jnp.pad(L0["W1"], (pad6, pad6))
    diag0 = L0["Wskip"][jnp.arange(_NSPEC), jnp.arange(_NSPEC)]
    d0p = jnp.pad(diag0, (pad6, (0, 0)))
    oh, table0, skip0 = _node0(spec.reshape(N, 1), w1p, d0p)
    parts0 = _sc_gather_scatter(table0, h_all, 0, 16, Np, snd, rcv)
    w2_0p = jnp.pad(L0["W2"], (pad6, (0, 0)))

    table1, skip1 = _node1(parts0[:N], parts0[Np:Np + N], skip0, oh,
                           w2_0p, L1["W1"], L1["Wskip"])
    parts1 = _sc_gather_scatter(table1, h_all, 16, 64, Np, snd, rcv)
    aep = jnp.pad(params["atom_energies"], pad6).reshape(16, 1)
    node_e = _node2(parts1[:N], parts1[Np:Np + N], skip1, oh, L1["W2"],
                    params["Wr"], aep)[:, 0]
    return node_e
